# baseline (speedup 1.0000x reference)
import functools

import jax
import jax.numpy as jnp
from jax.experimental import pallas as pl
from jax.experimental.pallas import tpu as pltpu

_INTERPRET = False


def _densify_body(key_ref, wval_ref, bidx_ref, bval_ref, dw_ref, bias_ref,
                  *, n_oc, nnz, n_taps):
    acc = jnp.zeros((n_oc, n_taps), jnp.float32)
    iota_j = jax.lax.broadcasted_iota(jnp.int32, (n_oc, n_taps), 1)
    for k in range(nnz):
        keyk = key_ref[:, k:k + 1]
        wk = wval_ref[:, k:k + 1]
        acc = acc + jnp.where(keyk == iota_j, wk, 0.0)
    dw_ref[:, :] = acc
    iota_oc = jax.lax.broadcasted_iota(jnp.int32, (n_oc, n_oc), 0)
    mask = bidx_ref[:, :] == iota_oc
    bias_ref[:, :] = jnp.sum(jnp.where(mask, bval_ref[:, :], 0.0), axis=1,
                             keepdims=True)


def _conv_body(imga_ref, imgb_ref, dw_ref, bias_ref, out_ref, x_ref, *, n_cin,
               ks, w_img, chunk, rows, w_out):
    bias = bias_ref[:, :]
    n_oc = out_ref.shape[1]
    n_cin_i, rows_i, w_i = imga_ref.shape[1], imga_ref.shape[2], imga_ref.shape[3]
    halo = x_ref.shape[1] - chunk
    x_ref[:, :chunk] = imga_ref[0, :, :, :].reshape(n_cin_i, chunk).astype(jnp.bfloat16)
    x_ref[:, chunk:] = imgb_ref[0, :, :, :].reshape(
        n_cin_i, imgb_ref.shape[2] * w_i)[:, :halo].astype(jnp.bfloat16)
    dw = dw_ref[:, :].astype(jnp.bfloat16)
    acc = jnp.zeros((n_oc, chunk), jnp.float32)
    for rc in range(ks * ks):
        r, cc = divmod(rc, ks)
        off = r * w_img + cc
        w_rc = dw[:, rc * n_cin:(rc + 1) * n_cin]
        x = x_ref[:, pl.ds(off, chunk)]
        acc = acc + jax.lax.dot_general(
            w_rc, x, (((1,), (0,)), ((), ())),
            preferred_element_type=jnp.float32)
    acc = acc + bias
    out_ref[0, :, :, :] = acc.reshape(n_oc, rows, w_img)[:, :, :w_out]


def kernel(images, weight_value, image_weight_index, image_range,
           filter_lengths, start_points, bias_index, bias_value):
    b, c_in, h, w = images.shape
    c_out = filter_lengths.shape[0]
    nnz = image_weight_index.shape[0] // c_out
    ks = 3
    h_out = (h - ks) + 1
    w_out = (w - ks) + 1
    hw = h * w
    n_taps = ks * ks * c_in

    idx = image_weight_index.reshape(c_out, nnz)
    cin = idx // hw
    rem = idx - cin * hw
    r = rem // w
    cc = rem - r * w
    key = (r * ks + cc) * c_in + cin

    wvals = weight_value.reshape(c_out, nnz)

    dense_w, bias2d = pl.pallas_call(
        functools.partial(_densify_body, n_oc=c_out, nnz=nnz, n_taps=n_taps),
        out_shape=(jax.ShapeDtypeStruct((c_out, n_taps), jnp.float32),
                   jax.ShapeDtypeStruct((c_out, 1), jnp.float32)),
        interpret=_INTERPRET,
    )(key, wvals, bias_index.reshape(1, c_out), bias_value.reshape(1, c_out))

    rows = 56
    n_chunks = -(-h_out // rows)
    chunk = rows * w
    halo_rows = 8
    halo = 896
    assert (ks - 1) * w + ks - 1 < halo <= halo_rows * w
    halo_img = jnp.concatenate(
        [images[:, :, rows * (j + 1):rows * (j + 1) + halo_rows]
         for j in range(n_chunks - 1)], axis=2)

    out = pl.pallas_call(
        functools.partial(_conv_body, n_cin=c_in, ks=ks, w_img=w, chunk=chunk,
                          rows=rows, w_out=w_out),
        grid=(b, n_chunks),
        in_specs=[
            pl.BlockSpec((1, c_in, rows, w), lambda i, j: (i, 0, j, 0)),
            pl.BlockSpec((1, c_in, halo_rows, w),
                         lambda i, j: (i, 0, jnp.minimum(j, n_chunks - 2), 0)),
            pl.BlockSpec((c_out, n_taps), lambda i, j: (0, 0)),
            pl.BlockSpec((c_out, 1), lambda i, j: (0, 0)),
        ],
        out_specs=pl.BlockSpec((1, c_out, rows, w_out), lambda i, j: (i, 0, j, 0)),
        out_shape=jax.ShapeDtypeStruct((b, c_out, h_out, w_out), jnp.float32),
        scratch_shapes=[pltpu.VMEM((c_in, chunk + halo), jnp.bfloat16)],
        interpret=_INTERPRET,
    )(images, halo_img, dense_w, bias2d)

    return out


# speedup vs baseline: 1.6927x; 1.6927x over previous
import functools

import jax
import jax.numpy as jnp
from jax.experimental import pallas as pl
from jax.experimental.pallas import tpu as pltpu

_INTERPRET = False


def _densify_body(key_ref, wval_ref, bidx_ref, bval_ref, dw_ref, bias_ref,
                  *, n_oc, nnz, n_taps):
    acc = jnp.zeros((n_oc, n_taps), jnp.float32)
    iota_j = jax.lax.broadcasted_iota(jnp.int32, (n_oc, n_taps), 1)
    for k in range(nnz):
        keyk = key_ref[:, k:k + 1]
        wk = wval_ref[:, k:k + 1]
        acc = acc + jnp.where(keyk == iota_j, wk, 0.0)
    dw_ref[:, :] = acc
    iota_oc = jax.lax.broadcasted_iota(jnp.int32, (n_oc, n_oc), 0)
    mask = bidx_ref[:, :] == iota_oc
    bias_ref[:, :] = jnp.sum(jnp.where(mask, bval_ref[:, :], 0.0), axis=1,
                             keepdims=True)


def _conv_body(imga_ref, imgb_ref, dw_ref, bias_ref, out_ref, x_ref, *, n_cin,
               ks, w_img, chunk, rows, w_out):
    bias = bias_ref[:, :]
    n_oc = out_ref.shape[2]
    n_cin_i, rows_i, w_i = imga_ref.shape[1], imga_ref.shape[2], imga_ref.shape[3]
    halo = x_ref.shape[1] - chunk
    x_ref[:, :chunk] = imga_ref[0, :, :, :].reshape(n_cin_i, chunk).astype(jnp.bfloat16)
    x_ref[:, chunk:] = imgb_ref[0, :, :, :].reshape(
        n_cin_i, imgb_ref.shape[2] * w_i)[:, :halo].astype(jnp.bfloat16)
    dw = dw_ref[:, :].astype(jnp.bfloat16)
    acc = jnp.zeros((n_oc, chunk), jnp.float32)
    for rc in range(ks * ks):
        r, cc = divmod(rc, ks)
        off = r * w_img + cc
        w_rc = dw[:, rc * n_cin:(rc + 1) * n_cin]
        x = x_ref[:, pl.ds(off, chunk)]
        acc = acc + jax.lax.dot_general(
            w_rc, x, (((1,), (0,)), ((), ())),
            preferred_element_type=jnp.float32)
    acc = acc + bias
    for i in range(rows):
        out_ref[0, i, :, :] = acc[:, i * w_img:i * w_img + w_out]


def kernel(images, weight_value, image_weight_index, image_range,
           filter_lengths, start_points, bias_index, bias_value):
    b, c_in, h, w = images.shape
    c_out = filter_lengths.shape[0]
    nnz = image_weight_index.shape[0] // c_out
    ks = 3
    h_out = (h - ks) + 1
    w_out = (w - ks) + 1
    hw = h * w
    n_taps = ks * ks * c_in

    idx = image_weight_index.reshape(c_out, nnz)
    cin = idx // hw
    rem = idx - cin * hw
    r = rem // w
    cc = rem - r * w
    key = (r * ks + cc) * c_in + cin

    wvals = weight_value.reshape(c_out, nnz)

    dense_w, bias2d = pl.pallas_call(
        functools.partial(_densify_body, n_oc=c_out, nnz=nnz, n_taps=n_taps),
        out_shape=(jax.ShapeDtypeStruct((c_out, n_taps), jnp.float32),
                   jax.ShapeDtypeStruct((c_out, 1), jnp.float32)),
        interpret=_INTERPRET,
    )(key, wvals, bias_index.reshape(1, c_out), bias_value.reshape(1, c_out))

    rows = 56
    n_chunks = -(-h_out // rows)
    chunk = rows * w
    halo_rows = 8
    halo = 896
    assert (ks - 1) * w + ks - 1 < halo <= halo_rows * w
    last_halo = h // halo_rows - 1

    out = pl.pallas_call(
        functools.partial(_conv_body, n_cin=c_in, ks=ks, w_img=w, chunk=chunk,
                          rows=rows, w_out=w_out),
        grid=(b, n_chunks),
        in_specs=[
            pl.BlockSpec((1, c_in, rows, w), lambda i, j: (i, 0, j, 0)),
            pl.BlockSpec((1, c_in, halo_rows, w),
                         lambda i, j: (i, 0, jnp.minimum(
                             (j + 1) * (rows // halo_rows), last_halo), 0)),
            pl.BlockSpec((c_out, n_taps), lambda i, j: (0, 0)),
            pl.BlockSpec((c_out, 1), lambda i, j: (0, 0)),
        ],
        out_specs=pl.BlockSpec((1, rows, c_out, w_out), lambda i, j: (i, j, 0, 0)),
        out_shape=jax.ShapeDtypeStruct((b, h_out, c_out, w_out), jnp.float32),
        scratch_shapes=[pltpu.VMEM((c_in, chunk + halo), jnp.bfloat16)],
        interpret=_INTERPRET,
    )(images, images, dense_w, bias2d)

    return out.transpose(0, 2, 1, 3)


# speedup vs baseline: 1.6957x; 1.0018x over previous
import functools

import jax
import jax.numpy as jnp
from jax.experimental import pallas as pl
from jax.experimental.pallas import tpu as pltpu

_INTERPRET = False


def _densify_body(key_ref, wval_ref, bidx_ref, bval_ref, dw_ref, bias_ref,
                  *, n_oc, nnz, n_taps):
    acc = jnp.zeros((n_oc, n_taps), jnp.float32)
    iota_j = jax.lax.broadcasted_iota(jnp.int32, (n_oc, n_taps), 1)
    for k in range(nnz):
        keyk = key_ref[:, k:k + 1]
        wk = wval_ref[:, k:k + 1]
        acc = acc + jnp.where(keyk == iota_j, wk, 0.0)
    dw_ref[:, :] = acc
    iota_oc = jax.lax.broadcasted_iota(jnp.int32, (n_oc, n_oc), 0)
    mask = bidx_ref[:, :] == iota_oc
    bias_ref[:, :] = jnp.sum(jnp.where(mask, bval_ref[:, :], 0.0), axis=1,
                             keepdims=True)


def _conv_body(imga_ref, imgb_ref, dw_ref, bias_ref, out_ref, x_ref, *, n_cin,
               ks, w_img, chunk, rows, w_out):
    bias = bias_ref[:, :]
    n_oc = out_ref.shape[2]
    n_cin_i, w_i = imga_ref.shape[1], imga_ref.shape[3]
    halo = x_ref.shape[1] - chunk
    x_ref[:, :chunk] = imga_ref[0, :, :, :].reshape(n_cin_i, chunk).astype(jnp.bfloat16)
    x_ref[:, chunk:] = imgb_ref[0, :, :, :].reshape(
        n_cin_i, imgb_ref.shape[2] * w_i)[:, :halo].astype(jnp.bfloat16)
    dw = dw_ref[:, :].astype(jnp.bfloat16)
    acc = jnp.zeros((n_oc, chunk), jnp.float32)
    for rc in range(ks * ks):
        r, cc = divmod(rc, ks)
        off = r * w_img + cc
        w_rc = dw[:, rc * n_cin:(rc + 1) * n_cin]
        x = x_ref[:, pl.ds(off, chunk)]
        acc = acc + jax.lax.dot_general(
            w_rc, x, (((1,), (0,)), ((), ())),
            preferred_element_type=jnp.float32)
    acc = acc + bias
    for i in range(rows):
        out_ref[0, i, :, :] = acc[:, i * w_img:i * w_img + w_out]


def kernel(images, weight_value, image_weight_index, image_range,
           filter_lengths, start_points, bias_index, bias_value):
    b, c_in, h, w = images.shape
    c_out = filter_lengths.shape[0]
    nnz = image_weight_index.shape[0] // c_out
    ks = 3
    h_out = (h - ks) + 1
    w_out = (w - ks) + 1
    hw = h * w
    n_taps = ks * ks * c_in

    idx = image_weight_index.reshape(c_out, nnz)
    cin = idx // hw
    rem = idx - cin * hw
    r = rem // w
    cc = rem - r * w
    key = (r * ks + cc) * c_in + cin

    wvals = weight_value.reshape(c_out, nnz)

    dense_w, bias2d = pl.pallas_call(
        functools.partial(_densify_body, n_oc=c_out, nnz=nnz, n_taps=n_taps),
        out_shape=(jax.ShapeDtypeStruct((c_out, n_taps), jnp.float32),
                   jax.ShapeDtypeStruct((c_out, 1), jnp.float32)),
        interpret=_INTERPRET,
    )(key, wvals, bias_index.reshape(1, c_out), bias_value.reshape(1, c_out))

    rows = 56
    n_chunks = -(-h_out // rows)
    chunk = rows * w
    halo_rows = 8
    halo = 896
    assert (ks - 1) * w + ks - 1 < halo <= halo_rows * w
    last_halo = h // halo_rows - 1

    out = pl.pallas_call(
        functools.partial(_conv_body, n_cin=c_in, ks=ks, w_img=w, chunk=chunk,
                          rows=rows, w_out=w_out),
        grid=(b, n_chunks),
        in_specs=[
            pl.BlockSpec((1, c_in, rows, w), lambda i, j: (i, 0, j, 0)),
            pl.BlockSpec((1, c_in, halo_rows, w),
                         lambda i, j: (i, 0, jnp.minimum(
                             (j + 1) * (rows // halo_rows), last_halo), 0)),
            pl.BlockSpec((c_out, n_taps), lambda i, j: (0, 0)),
            pl.BlockSpec((c_out, 1), lambda i, j: (0, 0)),
        ],
        out_specs=pl.BlockSpec((1, rows, c_out, w_out), lambda i, j: (i, j, 0, 0)),
        out_shape=jax.ShapeDtypeStruct((b, h_out, c_out, w_out), jnp.float32),
        scratch_shapes=[pltpu.VMEM((c_in, chunk + halo), jnp.bfloat16)],
        interpret=_INTERPRET,
    )(images, images, dense_w, bias2d)

    return out.transpose(0, 2, 1, 3)
